# Initial kernel scaffold; baseline (speedup 1.0000x reference)
import jax
import jax.numpy as jnp
from jax.experimental import pallas as pl


def kernel(X, emb):
    raise NotImplementedError("write your pallas kernel here")


# speedup vs baseline: 7.1633x; 7.1633x over previous
import functools
import jax
import jax.numpy as jnp
from jax.experimental import pallas as pl
from jax.experimental.pallas import tpu as pltpu

_NUM_NUCLEOTIDES = 4
_SEQ_LEN = 200
_EMBED_DIM = 64
_BATCH = 4096


def _pe_matrix():
    i_num = jnp.arange(0.0, _SEQ_LEN, dtype=jnp.float32).reshape(-1, 1)
    j_denom = jnp.power(
        10000.0, jnp.arange(0.0, _EMBED_DIM, 2.0, dtype=jnp.float32) / _EMBED_DIM
    )
    pe = jnp.zeros((_SEQ_LEN, _EMBED_DIM), dtype=jnp.float32)
    pe = pe.at[:, 0::2].set(jnp.sin(i_num / j_denom))
    pe = pe.at[:, 1::2].set(jnp.cos(i_num / j_denom))
    return pe


def _body(x_ref, emb_ref, pe_ref, o_ref):
    x = x_ref[...][:, :, None]
    emb = emb_ref[...]
    pe = pe_ref[...][None, :, :]
    e0 = emb[0][None, None, :]
    e1 = emb[1][None, None, :]
    e2 = emb[2][None, None, :]
    e3 = emb[3][None, None, :]
    picked = jnp.where(
        x < 2,
        jnp.where(x == 0, e0, e1),
        jnp.where(x == 2, e2, e3),
    )
    o_ref[...] = picked + pe


def _lookup_pe(X, emb, pe, block_b):
    grid = (X.shape[0] // block_b,)
    return pl.pallas_call(
        _body,
        grid=grid,
        in_specs=[
            pl.BlockSpec((block_b, _SEQ_LEN), lambda i: (i, 0)),
            pl.BlockSpec((_NUM_NUCLEOTIDES, _EMBED_DIM), lambda i: (0, 0)),
            pl.BlockSpec((_SEQ_LEN, _EMBED_DIM), lambda i: (0, 0)),
        ],
        out_specs=pl.BlockSpec((block_b, _SEQ_LEN, _EMBED_DIM), lambda i: (i, 0, 0)),
        out_shape=jax.ShapeDtypeStruct(
            (X.shape[0], _SEQ_LEN, _EMBED_DIM), jnp.float32
        ),
    )(X, emb, pe)


@jax.jit
def kernel(X, emb):
    X = X.astype(jnp.int32)
    pe = _pe_matrix()
    return _lookup_pe(X, emb, pe, block_b=128)
